# baseline (speedup 1.0000x reference)
import functools

import jax
import jax.numpy as jnp
from jax.experimental import pallas as pl
from jax.experimental.pallas import tpu as pltpu

_N = 10000
_D = 128
_HID = 128
_HEADS = 4
_HD = _HID // _HEADS
_NT = 3
_ET = 4

_BN = 1024


def _proj_body(x_ref, oh_ref, Ws_ref, bs_ref, Wd_ref, bd_ref, be_ref,
               Wec_ref, bec_ref, hs_ref, hd_ref, g_ref):
    x = x_ref[...]
    acc_s = jnp.zeros((x.shape[0], _HID), jnp.float32)
    acc_d = jnp.zeros((x.shape[0], _HID), jnp.float32)
    for t in range(_NT):
        m = oh_ref[:, t][:, None]
        acc_s = acc_s + m * (
            jax.lax.dot(x, Ws_ref[t], preferred_element_type=jnp.float32)
            + bs_ref[t][None, :])
        acc_d = acc_d + m * (
            jax.lax.dot(x, Wd_ref[t], preferred_element_type=jnp.float32)
            + bd_ref[t][None, :])
    hs_ref[...] = acc_s
    hd_ref[...] = acc_d
    cols = []
    for t in range(_ET):
        r = jnp.maximum(acc_s + be_ref[t][None, :], 0.0)
        cols.append(
            jax.lax.dot(r, Wec_ref[t][:, None],
                        preferred_element_type=jnp.float32)[:, 0]
            + bec_ref[t, 0])
    g = jnp.stack(cols, axis=-1)
    g_ref[...] = jnp.pad(g, ((0, 0), (0, 8 - _ET)))


def _projections(x, nt_onehot, W_src, b_src, W_dst, b_dst, b_edge, We_cls,
                 be_cls):
    n = x.shape[0]
    grid = (pl.cdiv(n, _BN),)
    full = lambda s: pl.BlockSpec(s, lambda i: (0,) * len(s))
    return pl.pallas_call(
        _proj_body,
        grid=grid,
        in_specs=[
            pl.BlockSpec((_BN, _D), lambda i: (i, 0)),
            pl.BlockSpec((_BN, 8), lambda i: (i, 0)),
            full((_NT, _D, _HID)),
            full((_NT, _HID)),
            full((_NT, _D, _HID)),
            full((_NT, _HID)),
            full((_ET, _HID)),
            full((_ET, _HID)),
            full((_ET, 1)),
        ],
        out_specs=[
            pl.BlockSpec((_BN, _HID), lambda i: (i, 0)),
            pl.BlockSpec((_BN, _HID), lambda i: (i, 0)),
            pl.BlockSpec((_BN, 8), lambda i: (i, 0)),
        ],
        out_shape=[
            jax.ShapeDtypeStruct((n, _HID), jnp.float32),
            jax.ShapeDtypeStruct((n, _HID), jnp.float32),
            jax.ShapeDtypeStruct((n, 8), jnp.float32),
        ],
    )(x, nt_onehot, W_src, b_src, W_dst, b_dst, b_edge, We_cls, be_cls)


def kernel(x, edge_index, node_type, edge_type, edge_attr, W_src, b_src,
           W_dst, b_dst, W_edge, b_edge, att, Wn_cls, bn_cls, We_cls, be_cls):
    n = x.shape[0]
    src = edge_index[0].astype(jnp.int32)
    dst = edge_index[1].astype(jnp.int32)
    nt = node_type.astype(jnp.int32)
    et = edge_type.astype(jnp.int32)

    nt_onehot = jax.nn.one_hot(nt, 8, dtype=jnp.float32)
    h_src, h_dst, g8 = _projections(x, nt_onehot, W_src, b_src, W_dst, b_dst,
                                    b_edge, We_cls[:, :, 0], be_cls)

    eemb = b_edge[et]
    m = jnp.take(h_src, src, axis=0) + eemb
    k = (m + jnp.take(h_dst, dst, axis=0)).reshape(-1, _HEADS, _HD)
    k = jax.nn.leaky_relu(k, 0.2)
    a = jnp.take(att, et, axis=0)
    scores = jnp.sum(k * a, axis=-1)
    smax = jax.ops.segment_max(scores, dst, num_segments=n)
    ex = jnp.exp(scores - jnp.take(smax, dst, axis=0))
    denom = jax.ops.segment_sum(ex, dst, num_segments=n)
    alpha = ex / (jnp.take(denom, dst, axis=0) + 1e-16)
    msg = m.reshape(-1, _HEADS, _HD) * alpha[:, :, None]
    x_out = jax.ops.segment_sum(msg, dst, num_segments=n).reshape(n, _HID)
    x_out = jax.nn.relu(x_out)

    x_cls = jnp.zeros((n, 1), jnp.float32)
    for t in range(_NT):
        mask = (nt == t)[:, None].astype(jnp.float32)
        x_cls = x_cls + mask * (x_out @ Wn_cls[t] + bn_cls[t])

    e_cls = g8[src, et][:, None]
    return (x_cls, e_cls)


# speedup vs baseline: 21.2699x; 21.2699x over previous
import functools

import jax
import jax.numpy as jnp
from jax import lax
from jax.experimental import pallas as pl
from jax.experimental.pallas import tpu as pltpu
from jax.experimental.pallas import tpu_sc as plsc

_NC = 2
_NS = 16
_NW = _NC * _NS

_N = 10000
_D = 128
_HID = 128
_HEADS = 4
_HD = _HID // _HEADS
_NT = 3
_ET = 4

_BN = 1024


def _proj_body(x_ref, oh_ref, Ws_ref, bs_ref, Wd_ref, bd_ref, be_ref,
               Wec_ref, bec_ref, hs_ref, hd_ref, g_ref):
    x = x_ref[...]
    acc_s = jnp.zeros((x.shape[0], _HID), jnp.float32)
    acc_d = jnp.zeros((x.shape[0], _HID), jnp.float32)
    for t in range(_NT):
        m = oh_ref[:, t][:, None]
        acc_s = acc_s + m * (
            jax.lax.dot(x, Ws_ref[t], preferred_element_type=jnp.float32)
            + bs_ref[t][None, :])
        acc_d = acc_d + m * (
            jax.lax.dot(x, Wd_ref[t], preferred_element_type=jnp.float32)
            + bd_ref[t][None, :])
    hs_ref[...] = acc_s
    hd_ref[...] = acc_d
    cols = []
    for t in range(_ET):
        r = jnp.maximum(acc_s + be_ref[t][None, :], 0.0)
        cols.append(
            jax.lax.dot(r, Wec_ref[t][:, None],
                        preferred_element_type=jnp.float32)[:, 0]
            + bec_ref[t, 0])
    g = jnp.stack(cols, axis=-1)
    g_ref[...] = jnp.pad(g, ((0, 0), (0, 8 - _ET)))


def _projections(x, nt_onehot, W_src, b_src, W_dst, b_dst, b_edge, We_cls,
                 be_cls):
    n = x.shape[0]
    grid = (pl.cdiv(n, _BN),)
    full = lambda s: pl.BlockSpec(s, lambda i: (0,) * len(s))
    return pl.pallas_call(
        _proj_body,
        grid=grid,
        in_specs=[
            pl.BlockSpec((_BN, _D), lambda i: (i, 0)),
            pl.BlockSpec((_BN, 8), lambda i: (i, 0)),
            full((_NT, _D, _HID)),
            full((_NT, _HID)),
            full((_NT, _D, _HID)),
            full((_NT, _HID)),
            full((_ET, _HID)),
            full((_ET, _HID)),
            full((_ET, 1)),
        ],
        out_specs=[
            pl.BlockSpec((_BN, _HID), lambda i: (i, 0)),
            pl.BlockSpec((_BN, _HID), lambda i: (i, 0)),
            pl.BlockSpec((_BN, 8), lambda i: (i, 0)),
        ],
        out_shape=[
            jax.ShapeDtypeStruct((n, _HID), jnp.float32),
            jax.ShapeDtypeStruct((n, _HID), jnp.float32),
            jax.ShapeDtypeStruct((n, 8), jnp.float32),
        ],
    )(x, nt_onehot, W_src, b_src, W_dst, b_dst, b_edge, We_cls, be_cls)


_NPAD = 10240
_T4 = _NPAD * _HEADS
_CH = 80
_CE = 2000


def _sc_mesh():
    return plsc.VectorSubcoreMesh(core_axis_name="c", subcore_axis_name="s",
                                  num_cores=_NC, num_subcores=_NS)


_SC_PARAMS = None


def _leaky(v):
    return jnp.maximum(v, 0.0) + 0.2 * jnp.minimum(v, 0.0)


def _k1_body(src_hbm, dst_hbm, et_hbm, hsrc_hbm, hdst_hbm, eatt_hbm,
             scores_hbm, smaxp_hbm,
             sidx_v, didx_v, etb_v, hsbuf, hdbuf, eatt_v, sbuf, smax_v, sem):
    E = src_hbm.shape[0]
    epw = E // _NW
    wid = lax.axis_index("c") * _NS + lax.axis_index("s")
    base = wid * epw
    pltpu.sync_copy(eatt_hbm, eatt_v)

    def init_body(j, _):
        smax_v[pl.ds(j * 16, 16)] = jnp.full((16,), -jnp.inf, jnp.float32)
        return _

    lax.fori_loop(0, _T4 // 16 + 1, init_body, 0)

    lane = lax.iota(jnp.int32, 16)
    lmask = lane < _HEADS

    def chunk_body(cidx, _):
        off = base + cidx * _CH
        pltpu.sync_copy(src_hbm.at[pl.ds(off, _CH)], sidx_v)
        pltpu.sync_copy(dst_hbm.at[pl.ds(off, _CH)], didx_v)
        pltpu.sync_copy(et_hbm.at[pl.ds(off, _CH)], etb_v)
        pltpu.async_copy(hsrc_hbm.at[sidx_v], hsbuf, sem).wait()
        pltpu.async_copy(hdst_hbm.at[didx_v], hdbuf, sem).wait()

        def edge_body(e, _):
            ev = jnp.zeros((16,), jnp.int32) + e
            et_e = plsc.load_gather(etb_v, [ev])[0]
            dv = plsc.load_gather(didx_v, [ev])
            eb = et_e * (2 * _HID)
            prods = []
            for j in range(8):
                s_ = (hsbuf[e, pl.ds(j * 16, 16)]
                      + hdbuf[e, pl.ds(j * 16, 16)]
                      + eatt_v[pl.ds(eb + j * 16, 16)])
                prods.append(
                    _leaky(s_) * eatt_v[pl.ds(eb + _HID + j * 16, 16)])
            shs = [jnp.sum(prods[2 * h] + prods[2 * h + 1])
                   for h in range(_HEADS)]
            sv = jnp.zeros((16,), jnp.float32) + shs[3]
            for h in range(_HEADS - 1):
                sv = jnp.where(lane == h, shs[h], sv)
            plsc.store_scatter(sbuf, [ev * _HEADS + lane], sv, mask=lmask)
            idxv = dv * _HEADS + lane
            cur = plsc.load_gather(smax_v, [idxv])
            plsc.store_scatter(smax_v, [idxv], jnp.maximum(cur, sv),
                               mask=lmask)
            return _

        lax.fori_loop(0, _CH, edge_body, 0)
        pltpu.sync_copy(sbuf, scores_hbm.at[pl.ds(off * _HEADS, _CH * _HEADS)])
        return _

    lax.fori_loop(0, epw // _CH, chunk_body, 0)
    pltpu.sync_copy(smax_v.at[pl.ds(0, _T4)], smaxp_hbm.at[wid])


def _k1_scores(src, dst, et, h_src, h_dst, eatt):
    E = src.shape[0]
    f = pl.kernel(
        _k1_body,
        out_type=[jax.ShapeDtypeStruct((E * _HEADS,), jnp.float32),
                  jax.ShapeDtypeStruct((_NW, _T4), jnp.float32)],
        mesh=_sc_mesh(),
        compiler_params=pltpu.CompilerParams(needs_layout_passes=False),
        scratch_types=[
            pltpu.VMEM((_CH,), jnp.int32),
            pltpu.VMEM((_CH,), jnp.int32),
            pltpu.VMEM((_CH,), jnp.int32),
            pltpu.VMEM((_CH, _HID), jnp.float32),
            pltpu.VMEM((_CH, _HID), jnp.float32),
            pltpu.VMEM((2 * _ET * _HID,), jnp.float32),
            pltpu.VMEM((_CH * _HEADS,), jnp.float32),
            pltpu.VMEM((_T4 + 16,), jnp.float32),
            pltpu.SemaphoreType.DMA,
        ],
    )
    return f(src, dst, et, h_src, h_dst, eatt)


def _reduce_body(op):
    def body(p_ref, o_ref):
        o_ref[...] = op(p_ref[...])
    return body


def _combine_tc(parts, op):
    bl = 5120
    return pl.pallas_call(
        _reduce_body(op),
        grid=(_T4 // bl,),
        in_specs=[pl.BlockSpec((_NW, bl), lambda i: (0, i))],
        out_specs=pl.BlockSpec((1, bl), lambda i: (0, i)),
        out_shape=jax.ShapeDtypeStruct((1, _T4), jnp.float32),
    )(parts)


def _k3_body(dst_hbm, scores_hbm, gmax_hbm, ex_hbm, denp_hbm,
             didx_v, sbuf, exbuf, smax_v, den_v):
    E = dst_hbm.shape[0]
    epw = E // _NW
    wid = lax.axis_index("c") * _NS + lax.axis_index("s")
    base = wid * epw
    pltpu.sync_copy(gmax_hbm, smax_v)

    def init_body(j, _):
        den_v[pl.ds(j * 16, 16)] = jnp.zeros((16,), jnp.float32)
        return _

    lax.fori_loop(0, _T4 // 16, init_body, 0)
    lane = lax.iota(jnp.int32, 16)
    eoff = lane >> 2
    hoff = lane & 3

    def chunk_body(cidx, _):
        off = base + cidx * _CE
        pltpu.sync_copy(dst_hbm.at[pl.ds(off, _CE)], didx_v)
        pltpu.sync_copy(scores_hbm.at[pl.ds(off * _HEADS, _CE * _HEADS)], sbuf)

        def vec_body(j, _):
            sc = sbuf[pl.ds(j * 16, 16)]
            d = plsc.load_gather(didx_v, [j * 4 + eoff])
            tgt = d * _HEADS + hoff
            sm = plsc.load_gather(smax_v, [tgt])
            ex = jnp.exp(sc - sm)
            exbuf[pl.ds(j * 16, 16)] = ex
            plsc.addupdate_scatter(den_v, [tgt], ex)
            return _

        lax.fori_loop(0, _CE * _HEADS // 16, vec_body, 0)
        pltpu.sync_copy(exbuf, ex_hbm.at[pl.ds(off * _HEADS, _CE * _HEADS)])
        return _

    lax.fori_loop(0, epw // _CE, chunk_body, 0)
    pltpu.sync_copy(den_v, denp_hbm.at[wid])


def _k3_exp_denom(dst, scores, gmax):
    E = dst.shape[0]
    f = pl.kernel(
        _k3_body,
        out_type=[jax.ShapeDtypeStruct((E * _HEADS,), jnp.float32),
                  jax.ShapeDtypeStruct((_NW, _T4), jnp.float32)],
        mesh=_sc_mesh(),
        compiler_params=pltpu.CompilerParams(needs_layout_passes=False),
        scratch_types=[
            pltpu.VMEM((_CE,), jnp.int32),
            pltpu.VMEM((_CE * _HEADS,), jnp.float32),
            pltpu.VMEM((_CE * _HEADS,), jnp.float32),
            pltpu.VMEM((_T4,), jnp.float32),
            pltpu.VMEM((_T4,), jnp.float32),
        ],
    )
    return f(dst, scores, gmax)


def _k45_body(dst_hbm, ex_hbm, invd_hbm, alpha_hbm,
              didx_v, exbuf, albuf, invd_v):
    E = dst_hbm.shape[0]
    epw = E // _NW
    wid = lax.axis_index("c") * _NS + lax.axis_index("s")
    base = wid * epw
    pltpu.sync_copy(invd_hbm, invd_v)
    lane = lax.iota(jnp.int32, 16)
    eoff = lane >> 2
    hoff = lane & 3

    def chunk_body(cidx, _):
        off = base + cidx * _CE
        pltpu.sync_copy(dst_hbm.at[pl.ds(off, _CE)], didx_v)
        pltpu.sync_copy(ex_hbm.at[pl.ds(off * _HEADS, _CE * _HEADS)], exbuf)

        def vec_body(j, _):
            ex = exbuf[pl.ds(j * 16, 16)]
            d = plsc.load_gather(didx_v, [j * 4 + eoff])
            tgt = d * _HEADS + hoff
            albuf[pl.ds(j * 16, 16)] = ex * plsc.load_gather(invd_v, [tgt])
            return _

        lax.fori_loop(0, _CE * _HEADS // 16, vec_body, 0)
        pltpu.sync_copy(albuf, alpha_hbm.at[pl.ds(off * _HEADS,
                                                  _CE * _HEADS)])
        return _

    lax.fori_loop(0, epw // _CE, chunk_body, 0)


def _k45_alpha(dst, ex, invd):
    E = dst.shape[0]
    f = pl.kernel(
        _k45_body,
        out_type=jax.ShapeDtypeStruct((E * _HEADS,), jnp.float32),
        mesh=_sc_mesh(),
        compiler_params=pltpu.CompilerParams(needs_layout_passes=False),
        scratch_types=[
            pltpu.VMEM((_CE,), jnp.int32),
            pltpu.VMEM((_CE * _HEADS,), jnp.float32),
            pltpu.VMEM((_CE * _HEADS,), jnp.float32),
            pltpu.VMEM((_T4,), jnp.float32),
        ],
    )
    return f(dst, ex, invd)


def _k5_body(src_hbm, dst_hbm, et_hbm, alpha_hbm, hsrc_hbm, eatt_hbm,
             xoutp_hbm,
             sidx_v, didx_v, etb_v, hsbuf, msgbuf, eatt_v, alb_v,
             zbuf, xout_sh, sem):
    E = src_hbm.shape[0]
    epw = E // _NW
    cid = lax.axis_index("c")
    sid = lax.axis_index("s")
    wid = cid * _NS + sid
    base = wid * epw
    rows_per_tile = _NPAD // _NS
    pltpu.sync_copy(eatt_hbm, eatt_v)

    def zrow(j, _):
        zbuf[j // 8, pl.ds((j % 8) * 16, 16)] = jnp.zeros((16,), jnp.float32)
        return _

    lax.fori_loop(0, 64 * (_HID // 16), zrow, 0)

    def zcopy(r, _):
        pltpu.sync_copy(
            zbuf, xout_sh.at[pl.ds(sid * rows_per_tile + r * 64, 64)])
        return _

    lax.fori_loop(0, rows_per_tile // 64, zcopy, 0)
    plsc.subcore_barrier()

    lane = lax.iota(jnp.int32, 16)

    def chunk_body(cidx, _):
        off = base + cidx * _CH
        pltpu.sync_copy(src_hbm.at[pl.ds(off, _CH)], sidx_v)
        pltpu.sync_copy(dst_hbm.at[pl.ds(off, _CH)], didx_v)
        pltpu.sync_copy(et_hbm.at[pl.ds(off, _CH)], etb_v)
        pltpu.sync_copy(alpha_hbm.at[pl.ds(off * _HEADS, _CH * _HEADS)],
                        alb_v.at[pl.ds(0, _CH * _HEADS)])
        pltpu.async_copy(hsrc_hbm.at[sidx_v], hsbuf, sem).wait()

        def edge_body(e, _):
            ev = jnp.zeros((16,), jnp.int32) + e
            et_e = plsc.load_gather(etb_v, [ev])[0]
            eb = et_e * (2 * _HID)
            av = plsc.load_gather(alb_v, [ev * _HEADS + lane])
            for j in range(8):
                m = (hsbuf[e, pl.ds(j * 16, 16)]
                     + eatt_v[pl.ds(eb + j * 16, 16)])
                msgbuf[e, pl.ds(j * 16, 16)] = m * av[j // 2]
            return _

        lax.fori_loop(0, _CH, edge_body, 0)
        pltpu.async_copy(msgbuf, xout_sh.at[didx_v], sem, add=True).wait()
        return _

    lax.fori_loop(0, epw // _CH, chunk_body, 0)
    plsc.subcore_barrier()
    pltpu.sync_copy(
        xout_sh.at[pl.ds(sid * rows_per_tile, rows_per_tile)],
        xoutp_hbm.at[cid].at[pl.ds(sid * rows_per_tile, rows_per_tile)])


def _k5_xout(src, dst, et, alpha, h_src, eatt):
    f = pl.kernel(
        _k5_body,
        out_type=jax.ShapeDtypeStruct((_NC, _NPAD, _HID), jnp.float32),
        mesh=_sc_mesh(),
        compiler_params=pltpu.CompilerParams(needs_layout_passes=False),
        scratch_types=[
            pltpu.VMEM((_CH,), jnp.int32),
            pltpu.VMEM((_CH,), jnp.int32),
            pltpu.VMEM((_CH,), jnp.int32),
            pltpu.VMEM((_CH, _HID), jnp.float32),
            pltpu.VMEM((_CH, _HID), jnp.float32),
            pltpu.VMEM((2 * _ET * _HID,), jnp.float32),
            pltpu.VMEM((_CH * _HEADS + 16,), jnp.float32),
            pltpu.VMEM((64, _HID), jnp.float32),
            pltpu.VMEM_SHARED((_NPAD, _HID), jnp.float32),
            pltpu.SemaphoreType.DMA,
        ],
    )
    return f(src, dst, et, alpha, h_src, eatt)


def _k7_body(xp_ref, oh_ref, Wn_ref, bn_ref, out_ref):
    xo = jnp.maximum(xp_ref[0] + xp_ref[1], 0.0)
    y = jax.lax.dot(xo, Wn_ref[...], preferred_element_type=jnp.float32)
    y = y + bn_ref[...][None, :]
    out_ref[...] = jnp.sum(oh_ref[...] * y, axis=1, keepdims=True)


def _k7_xcls(xout_parts, nt_onehot, Wn_all, bn_all):
    bn = 1024
    return pl.pallas_call(
        _k7_body,
        grid=(_NPAD // bn,),
        in_specs=[
            pl.BlockSpec((_NC, bn, _HID), lambda i: (0, i, 0)),
            pl.BlockSpec((bn, 8), lambda i: (i, 0)),
            pl.BlockSpec((_HID, 8), lambda i: (0, 0)),
            pl.BlockSpec((8,), lambda i: (0,)),
        ],
        out_specs=pl.BlockSpec((bn, 1), lambda i: (i, 0)),
        out_shape=jax.ShapeDtypeStruct((_NPAD, 1), jnp.float32),
    )(xout_parts, nt_onehot, Wn_all, bn_all)


def _ecls_body(g_hbm, src_hbm, et_hbm, out_hbm, g_v, src_v, et_v, out_v):
    wid = lax.axis_index("c") * _NS + lax.axis_index("s")
    epw = out_hbm.shape[0] // _NW
    base = wid * epw
    pltpu.sync_copy(g_hbm, g_v)
    ch = src_v.shape[0]

    def chunk_body(c, _):
        off = base + c * ch
        pltpu.sync_copy(src_hbm.at[pl.ds(off, ch)], src_v)
        pltpu.sync_copy(et_hbm.at[pl.ds(off, ch)], et_v)

        def vec_body(j, _):
            s = src_v[pl.ds(j * 16, 16)]
            e = et_v[pl.ds(j * 16, 16)]
            idx = s * 4 + e
            out_v[pl.ds(j * 16, 16)] = plsc.load_gather(g_v, [idx])
            return _

        lax.fori_loop(0, ch // 16, vec_body, 0, unroll=4)
        pltpu.sync_copy(out_v, out_hbm.at[pl.ds(off, ch)])
        return _

    lax.fori_loop(0, epw // ch, chunk_body, 0)


def _ecls_gather(g4flat, src, et):
    E = src.shape[0]
    ch = 2000
    body = functools.partial(
        pl.kernel,
        out_type=jax.ShapeDtypeStruct((E,), jnp.float32),
        mesh=plsc.VectorSubcoreMesh(core_axis_name="c", subcore_axis_name="s",
                                    num_cores=_NC, num_subcores=_NS),
        compiler_params=pltpu.CompilerParams(needs_layout_passes=False),
        scratch_types=[
            pltpu.VMEM((g4flat.shape[0],), jnp.float32),
            pltpu.VMEM((ch,), jnp.int32),
            pltpu.VMEM((ch,), jnp.int32),
            pltpu.VMEM((ch,), jnp.float32),
        ],
    )(_ecls_body)
    return body(g4flat, src, et)


def kernel(x, edge_index, node_type, edge_type, edge_attr, W_src, b_src,
           W_dst, b_dst, W_edge, b_edge, att, Wn_cls, bn_cls, We_cls, be_cls):
    n = x.shape[0]
    src = edge_index[0].astype(jnp.int32)
    dst = edge_index[1].astype(jnp.int32)
    nt = node_type.astype(jnp.int32)
    et = edge_type.astype(jnp.int32)

    nt_onehot = jax.nn.one_hot(nt, 8, dtype=jnp.float32)
    h_src, h_dst, g8 = _projections(x, nt_onehot, W_src, b_src, W_dst, b_dst,
                                    b_edge, We_cls[:, :, 0], be_cls)

    eatt = jnp.concatenate(
        [b_edge, att.reshape(_ET, _HID)], axis=1).reshape(-1)
    scores, smax_part = _k1_scores(src, dst, et, h_src, h_dst, eatt)
    gmax = _combine_tc(
        smax_part, lambda p: jnp.max(p, axis=0, keepdims=True)).reshape(-1)
    ex, den_part = _k3_exp_denom(dst, scores, gmax)
    invd = _combine_tc(
        den_part,
        lambda p: 1.0 / (jnp.sum(p, axis=0, keepdims=True) + 1e-16),
    ).reshape(-1)
    alpha = _k45_alpha(dst, ex, invd)
    xout_parts = _k5_xout(src, dst, et, alpha, h_src, eatt)

    ohp = jnp.pad(nt_onehot, ((0, _NPAD - n), (0, 0)))
    Wn_all = jnp.pad(Wn_cls[:, :, 0].T, ((0, 0), (0, 8 - _NT)))
    bn_all = jnp.pad(bn_cls[:, 0], (0, 8 - _NT))
    x_cls = _k7_xcls(xout_parts, ohp, Wn_all, bn_all)[:n]

    g4flat = g8[:, :_ET].reshape(-1)
    e_cls = _ecls_gather(g4flat, src, et)[:, None]
    return (x_cls, e_cls)
